# Initial kernel scaffold; baseline (speedup 1.0000x reference)
import functools
import math

import jax
import jax.numpy as jnp
from jax import lax
from jax.experimental import pallas as pl
from jax.experimental.pallas import tpu as pltpu
from jax.experimental.pallas import tpu_sc as plsc

_D = 128
_SCALE = math.sqrt(_D)

_NC = 2
_NS = 16
_NW = _NC * _NS

_CHUNK = 256


def _scale_body(lut_ref, out_ref):
    out_ref[...] = lut_ref[...] * _SCALE


def _scale_table(lut):
    v = lut.shape[0]
    blk = 2500
    return pl.pallas_call(
        _scale_body,
        out_shape=jax.ShapeDtypeStruct(lut.shape, lut.dtype),
        grid=(v // blk,),
        in_specs=[pl.BlockSpec((blk, _D), lambda i: (i, 0))],
        out_specs=pl.BlockSpec((blk, _D), lambda i: (i, 0)),
    )(lut)


def _make_gather(n_rows):
    assert n_rows % (_NW * _CHUNK) == 0
    b_per_w = n_rows // _NW
    n_chunks = b_per_w // _CHUNK
    mesh = plsc.VectorSubcoreMesh(core_axis_name="c", subcore_axis_name="s")

    @functools.partial(
        pl.kernel,
        out_type=jax.ShapeDtypeStruct((n_rows, _D), jnp.float32),
        mesh=mesh,
        scratch_types=[
            pltpu.VMEM((2, _CHUNK), jnp.int32),
            pltpu.VMEM((2, _CHUNK, _D), jnp.float32),
            pltpu.SemaphoreType.DMA((2,)),
        ],
    )
    def gather(table_hbm, idx_hbm, out_hbm, idx_v, rows_v, sems):
        wid = lax.axis_index("s") * _NC + lax.axis_index("c")
        base = wid * b_per_w

        def fire(chunk, slot):
            off = base + chunk * _CHUNK
            pltpu.sync_copy(idx_hbm.at[pl.ds(off, _CHUNK)], idx_v.at[slot])
            pltpu.async_copy(table_hbm.at[idx_v.at[slot]], rows_v.at[slot],
                             sems.at[slot])

        def drain(chunk, slot):
            off = base + chunk * _CHUNK
            pltpu.make_async_copy(table_hbm.at[idx_v.at[slot]],
                                  rows_v.at[slot], sems.at[slot]).wait()
            pltpu.sync_copy(rows_v.at[slot], out_hbm.at[pl.ds(off, _CHUNK)])

        fire(0, 0)

        def body(i, _):
            slot = lax.rem(i, 2)
            nslot = lax.rem(i + 1, 2)

            @pl.when(i + 1 < n_chunks)
            def _():
                fire(i + 1, nslot)

            drain(i, slot)
            return ()

        lax.fori_loop(0, n_chunks, body, (), unroll=2)

    return gather


_gather = _make_gather(4096 * 200)


def kernel(x, lut):
    b, s = x.shape
    idx = x.reshape(b * s).astype(jnp.int32)
    scaled = _scale_table(lut)
    out = _gather(scaled, idx)
    return out.reshape(b, s, _D)


# speedup vs baseline: 7.4888x; 7.4888x over previous
import functools
import math

import jax
import jax.numpy as jnp
from jax import lax
from jax.experimental import pallas as pl
from jax.experimental.pallas import tpu as pltpu
from jax.experimental.pallas import tpu_sc as plsc

_D = 128
_SCALE = math.sqrt(_D)

_NC = 2
_NS = 16
_NW = _NC * _NS

_CHUNK = 128


def _scale_body(lut_ref, out_ref):
    out_ref[...] = lut_ref[...] * _SCALE


def _scale_table(lut):
    v = lut.shape[0]
    blk = 4000
    return pl.pallas_call(
        _scale_body,
        out_shape=jax.ShapeDtypeStruct(lut.shape, lut.dtype),
        grid=(v // blk,),
        in_specs=[pl.BlockSpec((blk, _D), lambda i: (i, 0))],
        out_specs=pl.BlockSpec((blk, _D), lambda i: (i, 0)),
    )(lut)


def _make_gather(n_rows):
    assert n_rows % (_NW * _CHUNK * 2) == 0
    b_per_w = n_rows // _NW
    n_chunks = b_per_w // _CHUNK
    mesh = plsc.VectorSubcoreMesh(core_axis_name="c", subcore_axis_name="s")

    @functools.partial(
        pl.kernel,
        out_type=jax.ShapeDtypeStruct((n_rows, _D), jnp.float32),
        mesh=mesh,
        scratch_types=[
            pltpu.VMEM((2, _CHUNK), jnp.int32),
            pltpu.VMEM((2, _CHUNK, _D), jnp.float32),
            pltpu.SemaphoreType.DMA,
            pltpu.SemaphoreType.DMA,
        ],
    )
    def gather(table_hbm, idx_hbm, out_hbm, idx_v, rows_v, sem0, sem1):
        wid = lax.axis_index("s") * _NC + lax.axis_index("c")
        base = wid * b_per_w
        sems = (sem0, sem1)

        def fire(chunk, slot):
            off = base + chunk * _CHUNK
            pltpu.sync_copy(idx_hbm.at[pl.ds(off, _CHUNK)], idx_v.at[slot])
            pltpu.async_copy(table_hbm.at[idx_v.at[slot]], rows_v.at[slot],
                             sems[slot])

        def drain(chunk, slot):
            off = base + chunk * _CHUNK
            pltpu.make_async_copy(table_hbm.at[idx_v.at[slot]],
                                  rows_v.at[slot], sems[slot]).wait()
            pltpu.sync_copy(rows_v.at[slot], out_hbm.at[pl.ds(off, _CHUNK)])

        fire(0, 0)

        def body(g, _):
            i = g * 2

            fire(i + 1, 1)
            drain(i, 0)

            @pl.when(i + 2 < n_chunks)
            def _():
                fire(i + 2, 0)

            drain(i + 1, 1)
            return ()

        lax.fori_loop(0, n_chunks // 2, body, ())

    return gather


_gather = _make_gather(4096 * 200)


def kernel(x, lut):
    b, s = x.shape
    idx = x.reshape(b * s).astype(jnp.int32)
    scaled = _scale_table(lut)
    out = _gather(scaled, idx)
    return out.reshape(b, s, _D)


# speedup vs baseline: 8.2876x; 1.1067x over previous
import functools
import math

import jax
import jax.numpy as jnp
from jax import lax
from jax.experimental import pallas as pl
from jax.experimental.pallas import tpu as pltpu
from jax.experimental.pallas import tpu_sc as plsc

_D = 128
_SCALE = math.sqrt(_D)

_NC = 2
_NS = 16
_NW = _NC * _NS

_CHUNK = 128
_NBUF = 4


def _scale_body(lut_ref, out_ref):
    out_ref[...] = lut_ref[...] * _SCALE


def _scale_table(lut):
    v = lut.shape[0]
    blk = 4000
    return pl.pallas_call(
        _scale_body,
        out_shape=jax.ShapeDtypeStruct(lut.shape, lut.dtype),
        grid=(v // blk,),
        in_specs=[pl.BlockSpec((blk, _D), lambda i: (i, 0))],
        out_specs=pl.BlockSpec((blk, _D), lambda i: (i, 0)),
    )(lut)


def _make_gather(n_rows):
    assert n_rows % (_NW * _CHUNK * _NBUF) == 0
    b_per_w = n_rows // _NW
    n_chunks = b_per_w // _CHUNK
    mesh = plsc.VectorSubcoreMesh(core_axis_name="c", subcore_axis_name="s")

    @functools.partial(
        pl.kernel,
        out_type=jax.ShapeDtypeStruct((n_rows, _D), jnp.float32),
        mesh=mesh,
        scratch_types=[
            pltpu.VMEM((n_chunks, _CHUNK), jnp.int32),
            pltpu.VMEM((_NBUF, _CHUNK, _D), jnp.float32),
            [pltpu.SemaphoreType.DMA] * _NBUF,
        ],
    )
    def gather(table_hbm, idx_hbm, out_hbm, idx_v, rows_v, sems):
        wid = lax.axis_index("s") * _NC + lax.axis_index("c")
        base = wid * b_per_w

        pltpu.sync_copy(idx_hbm.at[wid], idx_v)

        def fire(chunk, slot):
            pltpu.async_copy(table_hbm.at[idx_v.at[chunk]], rows_v.at[slot],
                             sems[slot])

        def drain(chunk, slot):
            pltpu.make_async_copy(table_hbm.at[idx_v.at[chunk]],
                                  rows_v.at[slot], sems[slot]).wait()
            off = base + chunk * _CHUNK
            pltpu.sync_copy(rows_v.at[slot], out_hbm.at[pl.ds(off, _CHUNK)])

        for b in range(_NBUF - 1):
            fire(b, b)

        def body(g, _):
            i = g * _NBUF
            for b in range(_NBUF):
                nxt = i + b + _NBUF - 1

                @pl.when(nxt < n_chunks)
                def _():
                    fire(nxt, (b + _NBUF - 1) % _NBUF)

                drain(i + b, b)
            return ()

        lax.fori_loop(0, n_chunks // _NBUF, body, ())

    return gather


_gather = _make_gather(4096 * 200)


def kernel(x, lut):
    b, s = x.shape
    n = b * s
    idx = x.reshape(_NW, n // (_NW * _CHUNK), _CHUNK).astype(jnp.int32)
    scaled = _scale_table(lut)
    out = _gather(scaled, idx)
    return out.reshape(b, s, _D)


# speedup vs baseline: 8.2947x; 1.0009x over previous
import functools
import math

import jax
import jax.numpy as jnp
from jax import lax
from jax.experimental import pallas as pl
from jax.experimental.pallas import tpu as pltpu
from jax.experimental.pallas import tpu_sc as plsc

_D = 128
_SCALE = math.sqrt(_D)

_NC = 2
_NS = 16
_NW = _NC * _NS

_CHUNK = 128
_NBUF = 5


def _scale_body(lut_ref, out_ref):
    out_ref[...] = lut_ref[...] * _SCALE


def _scale_table(lut):
    v = lut.shape[0]
    blk = 4000
    return pl.pallas_call(
        _scale_body,
        out_shape=jax.ShapeDtypeStruct(lut.shape, lut.dtype),
        grid=(v // blk,),
        in_specs=[pl.BlockSpec((blk, _D), lambda i: (i, 0))],
        out_specs=pl.BlockSpec((blk, _D), lambda i: (i, 0)),
    )(lut)


def _make_gather(n_rows):
    assert n_rows % (_NW * _CHUNK * _NBUF) == 0, n_rows
    b_per_w = n_rows // _NW
    n_chunks = b_per_w // _CHUNK
    mesh = plsc.VectorSubcoreMesh(core_axis_name="c", subcore_axis_name="s")

    @functools.partial(
        pl.kernel,
        out_type=jax.ShapeDtypeStruct((n_rows, _D), jnp.float32),
        mesh=mesh,
        scratch_types=[
            pltpu.VMEM((n_chunks, _CHUNK), jnp.int32),
            pltpu.VMEM((_NBUF, _CHUNK, _D), jnp.float32),
            [pltpu.SemaphoreType.DMA] * _NBUF,
        ],
    )
    def gather(table_hbm, idx_hbm, out_hbm, idx_v, rows_v, sems):
        wid = lax.axis_index("s") * _NC + lax.axis_index("c")
        base = wid * b_per_w

        pltpu.sync_copy(idx_hbm.at[wid], idx_v)

        def fire(chunk, slot):
            pltpu.async_copy(table_hbm.at[idx_v.at[chunk]], rows_v.at[slot],
                             sems[slot])

        def drain(chunk, slot):
            pltpu.make_async_copy(table_hbm.at[idx_v.at[chunk]],
                                  rows_v.at[slot], sems[slot]).wait()
            off = base + chunk * _CHUNK
            pltpu.sync_copy(rows_v.at[slot], out_hbm.at[pl.ds(off, _CHUNK)])

        for b in range(_NBUF - 1):
            fire(b, b)

        def body(g, _):
            i = g * _NBUF
            for b in range(_NBUF):
                nxt = i + b + _NBUF - 1

                @pl.when(nxt < n_chunks)
                def _():
                    fire(nxt, (b + _NBUF - 1) % _NBUF)

                drain(i + b, b)
            return ()

        lax.fori_loop(0, n_chunks // _NBUF, body, ())

    return gather


_gather = _make_gather(4096 * 200)


def kernel(x, lut):
    b, s = x.shape
    n = b * s
    idx = x.reshape(_NW, n // (_NW * _CHUNK), _CHUNK).astype(jnp.int32)
    scaled = _scale_table(lut)
    out = _gather(scaled, idx)
    return out.reshape(b, s, _D)


# speedup vs baseline: 9.0571x; 1.0919x over previous
import functools
import math

import jax
import jax.numpy as jnp
from jax import lax
from jax.experimental import pallas as pl
from jax.experimental.pallas import tpu as pltpu
from jax.experimental.pallas import tpu_sc as plsc

_D = 128
_SCALE = math.sqrt(_D)

_NC = 2
_NS = 16
_NW = _NC * _NS

_CHUNK = 128
_NBUF = 5
_L = 16


def _make_gather(n_rows):
    assert n_rows % (_NW * _CHUNK * _NBUF) == 0, n_rows
    b_per_w = n_rows // _NW
    n_chunks = b_per_w // _CHUNK
    mesh = plsc.VectorSubcoreMesh(core_axis_name="c", subcore_axis_name="s")

    @functools.partial(
        pl.kernel,
        out_type=jax.ShapeDtypeStruct((n_rows, _D), jnp.float32),
        mesh=mesh,
        scratch_types=[
            pltpu.VMEM((n_chunks, _CHUNK), jnp.int32),
            pltpu.VMEM((_NBUF, _CHUNK, _D), jnp.float32),
            [pltpu.SemaphoreType.DMA] * _NBUF,
        ],
    )
    def gather(table_hbm, idx_hbm, out_hbm, idx_v, rows_v, sems):
        wid = lax.axis_index("s") * _NC + lax.axis_index("c")
        base = wid * b_per_w

        pltpu.sync_copy(idx_hbm.at[wid], idx_v)

        def fire(chunk, slot):
            pltpu.async_copy(table_hbm.at[idx_v.at[chunk]], rows_v.at[slot],
                             sems[slot])

        def drain(chunk, slot):
            pltpu.make_async_copy(table_hbm.at[idx_v.at[chunk]],
                                  rows_v.at[slot], sems[slot]).wait()

            def scale_row(r, _):
                for j in range(_D // _L):
                    sl = pl.ds(j * _L, _L)
                    rows_v[slot, r, sl] = rows_v[slot, r, sl] * _SCALE
                return ()

            lax.fori_loop(0, _CHUNK, scale_row, (), unroll=2)
            off = base + chunk * _CHUNK
            pltpu.sync_copy(rows_v.at[slot], out_hbm.at[pl.ds(off, _CHUNK)])

        for b in range(_NBUF - 1):
            fire(b, b)

        def body(g, _):
            i = g * _NBUF
            for b in range(_NBUF):
                nxt = i + b + _NBUF - 1

                @pl.when(nxt < n_chunks)
                def _():
                    fire(nxt, (b + _NBUF - 1) % _NBUF)

                drain(i + b, b)
            return ()

        lax.fori_loop(0, n_chunks // _NBUF, body, ())

    return gather


_gather = _make_gather(4096 * 200)


def kernel(x, lut):
    b, s = x.shape
    n = b * s
    idx = x.reshape(_NW, n // (_NW * _CHUNK), _CHUNK).astype(jnp.int32)
    out = _gather(lut, idx)
    return out.reshape(b, s, _D)


# speedup vs baseline: 9.1249x; 1.0075x over previous
import functools
import math

import jax
import jax.numpy as jnp
from jax import lax
from jax.experimental import pallas as pl
from jax.experimental.pallas import tpu as pltpu
from jax.experimental.pallas import tpu_sc as plsc

_D = 128
_SCALE = math.sqrt(_D)

_NC = 2
_NS = 16
_NW = _NC * _NS

_CHUNK = 128
_NBUF = 5
_L = 16


def _make_gather(n_rows):
    assert n_rows % (_NW * _CHUNK * _NBUF) == 0, n_rows
    b_per_w = n_rows // _NW
    n_chunks = b_per_w // _CHUNK
    mesh = plsc.VectorSubcoreMesh(core_axis_name="c", subcore_axis_name="s")

    @functools.partial(
        pl.kernel,
        out_type=jax.ShapeDtypeStruct((n_rows, _D), jnp.float32),
        mesh=mesh,
        scratch_types=[
            pltpu.VMEM((n_chunks, _CHUNK), jnp.int32),
            pltpu.VMEM((_NBUF, _CHUNK, _D), jnp.float32),
            [pltpu.SemaphoreType.DMA] * _NBUF,
            [pltpu.SemaphoreType.DMA] * _NBUF,
        ],
    )
    def gather(table_hbm, idx_hbm, out_hbm, idx_v, rows_v, gsems, wsems):
        wid = lax.axis_index("s") * _NC + lax.axis_index("c")
        base = wid * b_per_w

        pltpu.sync_copy(idx_hbm.at[wid], idx_v)

        def fire(chunk, slot):
            pltpu.async_copy(table_hbm.at[idx_v.at[chunk]], rows_v.at[slot],
                             gsems[slot])

        def wb_copy(chunk, slot):
            off = base + chunk * _CHUNK
            return pltpu.make_async_copy(
                rows_v.at[slot], out_hbm.at[pl.ds(off, _CHUNK)], wsems[slot])

        def drain(chunk, slot):
            pltpu.make_async_copy(table_hbm.at[idx_v.at[chunk]],
                                  rows_v.at[slot], gsems[slot]).wait()

            def scale_row(r, _):
                for j in range(_D // _L):
                    sl = pl.ds(j * _L, _L)
                    rows_v[slot, r, sl] = rows_v[slot, r, sl] * _SCALE
                return ()

            lax.fori_loop(0, _CHUNK, scale_row, (), unroll=2)
            wb_copy(chunk, slot).start()

        for b in range(_NBUF - 1):
            fire(b, b)

        def body(g, _):
            i = g * _NBUF
            for b in range(_NBUF):
                nxt = i + b + _NBUF - 1
                slot_n = (b + _NBUF - 1) % _NBUF

                @pl.when(nxt < n_chunks)
                def _():
                    @pl.when(nxt >= _NBUF)
                    def _():
                        wb_copy(nxt - _NBUF, slot_n).wait()

                    fire(nxt, slot_n)

                drain(i + b, b)
            return ()

        lax.fori_loop(0, n_chunks // _NBUF, body, ())

        for b in range(_NBUF):
            wb_copy(n_chunks - _NBUF + b, b).wait()

    return gather


_gather = _make_gather(4096 * 200)


def kernel(x, lut):
    b, s = x.shape
    n = b * s
    idx = x.reshape(_NW, n // (_NW * _CHUNK), _CHUNK).astype(jnp.int32)
    out = _gather(lut, idx)
    return out.reshape(b, s, _D)
